# baseline (speedup 1.0000x reference)
import functools

import jax
import jax.numpy as jnp
from jax import lax
from jax.experimental import pallas as pl
from jax.experimental.pallas import tpu as pltpu
from jax.experimental.pallas import tpu_sc as plsc

N = 10000
E = 320000
L = 512
HID = 128
NUM_GRAPHS = 1000

NP = 10240
GP = 1024
TN = 256
NC, NS = 2, 16
NW = NC * NS


def _make_sc_scatter(T, Ep, ACC, CH):
  assert Ep % (NW * CH) == 0 and ACC % NS == 0
  nch = Ep // (NW * CH)
  per_w = Ep // NW
  zr = ACC // NS
  mesh = plsc.VectorSubcoreMesh(core_axis_name="c", subcore_axis_name="s")

  @functools.partial(
      pl.kernel, mesh=mesh,
      out_type=jax.ShapeDtypeStruct((2 * ACC, HID), jnp.float32),
      scratch_types=[
          pltpu.VMEM((CH,), jnp.int32),
          pltpu.VMEM((CH,), jnp.int32),
          pltpu.VMEM((CH, HID), jnp.float32),
          pltpu.VMEM_SHARED((ACC, HID), jnp.float32),
          pltpu.SemaphoreType.DMA,
      ],
  )
  def k(table, srci, dsti, zro, out, sidx_v, didx_v, rows_v, acc_sh, sem):
    cid = lax.axis_index("c")
    sid = lax.axis_index("s")
    pltpu.sync_copy(zro.at[pl.ds(sid * zr, zr)], acc_sh.at[pl.ds(sid * zr, zr)])
    plsc.subcore_barrier()
    wid = sid * NC + cid
    base0 = wid * per_w

    def body(i, _):
      base = base0 + i * CH
      pltpu.sync_copy(srci.at[pl.ds(base, CH)], sidx_v)
      pltpu.async_copy(table.at[sidx_v], rows_v, sem).wait()
      pltpu.sync_copy(dsti.at[pl.ds(base, CH)], didx_v)
      pltpu.sync_copy(rows_v, acc_sh.at[didx_v], add=True)
      return 0

    lax.fori_loop(0, nch, body, 0)
    plsc.subcore_barrier()
    pltpu.sync_copy(acc_sh.at[pl.ds(sid * zr, zr)],
                    out.at[pl.ds(cid * ACC + sid * zr, zr)])

  return k


TE = 32


def _enc_body(xq_ref, w1_ref, b1_ref, w2_ref, b2_ref, lw_ref, lb_ref, o_ref):
  xq = xq_ref[...]
  w1 = w1_ref[...]
  h1e = jnp.zeros((TE, 16, 128), jnp.float32)
  h1o = jnp.zeros((TE, 16, 128), jnp.float32)
  for kk in range(7):
    wk = w1[kk][None]
    h1e = h1e + xq[:, kk % 4, (kk // 4):(kk // 4) + 128][:, None, :] * wk
    ko = kk + 2
    h1o = h1o + xq[:, ko % 4, (ko // 4):(ko // 4) + 128][:, None, :] * wk
  b1 = b1_ref[...][None]
  h1e = jnp.maximum(h1e + b1, 0.0)
  h1o = jnp.maximum(h1o + b1, 0.0)
  z1 = jnp.zeros((TE, 16, 1), jnp.float32)
  h1pe = jnp.concatenate([z1, h1e, z1], axis=2)
  h1po = jnp.concatenate([z1, h1o, z1], axis=2)
  w2 = w2_ref[...]
  acc = jnp.zeros((TE, 32, 128), jnp.float32)
  for kk in range(5):
    off = kk // 2 if kk % 2 == 0 else (kk - 1) // 2
    hp = h1pe if kk % 2 == 0 else h1po
    for ii in range(16):
      hs = hp[:, ii, off:off + 128]
      acc = acc + hs[:, None, :] * w2[kk * 16 + ii][None]
  acc = jnp.maximum(acc + b2_ref[...][None], 0.0)
  m = jnp.mean(acc, axis=2)
  o_ref[...] = m @ lw_ref[...] + lb_ref[...][0][None, :]


def _pre_body(h_ref, w_ref, a0_ref, a1_ref, o_ref):
  dis = lax.rsqrt(1.0 + a0_ref[...] + a1_ref[...])
  o_ref[...] = dis * (h_ref[...] @ w_ref[...])


def _post_body(s0_ref, s1_ref, hws_ref, a0_ref, a1_ref, b_ref, o_ref):
  dis = lax.rsqrt(1.0 + a0_ref[...] + a1_ref[...])
  v = dis * (s0_ref[...] + s1_ref[...] + hws_ref[...]) + b_ref[...][0][None, :]
  o_ref[...] = jnp.maximum(v, 0.0)


def _final_body(p0_ref, p1_ref, c0_ref, c1_ref, fw_ref, fb_ref, o_ref):
  cnt = jnp.maximum(c0_ref[...] + c1_ref[...], 1.0)
  pooled = (p0_ref[...] + p1_ref[...]) / cnt
  o_ref[...] = pooled @ fw_ref[...] + fb_ref[...][0][None, :]


def _row_spec(d):
  return pl.BlockSpec((TN, d), lambda i: (i, 0))


def _full_spec(shape):
  return pl.BlockSpec(shape, lambda i: tuple(0 for _ in shape))


def kernel(x, edge_index, batch, conv1_w, conv1_b, conv2_w, conv2_b,
           lin_w, lin_b, g1_w, g1_b, g2_w, g2_b, fc_w, fc_b):
  f32 = jnp.float32
  grid = NP // TN

  x_p = jnp.concatenate([x, jnp.zeros((NP - N, L), f32)], axis=0)
  xpad = jnp.pad(x_p, ((0, 0), (3, 5)))
  xq = jnp.transpose(xpad.reshape(NP, 130, 4), (0, 2, 1))
  w1 = jnp.broadcast_to(conv1_w.reshape(16, 7).T[:, :, None], (7, 16, 128))
  w2 = jnp.broadcast_to(
      jnp.transpose(conv2_w, (2, 1, 0)).reshape(80, 32)[:, :, None],
      (80, 32, 128))
  b1 = jnp.broadcast_to(conv1_b[:, None], (16, 128))
  b2 = jnp.broadcast_to(conv2_b[:, None], (32, 128))
  lb = lin_b.reshape(1, HID)
  g1b = g1_b.reshape(1, HID)
  g2b = g2_b.reshape(1, HID)
  fw = jnp.zeros((HID, HID), f32).at[:, :fc_w.shape[1]].set(fc_w)
  fb = jnp.zeros((1, HID), f32).at[0, :fc_b.shape[0]].set(fc_b)

  EP = 323584
  src_p = jnp.concatenate([edge_index[0], jnp.zeros((EP - E,), jnp.int32)])
  dst_p = jnp.concatenate([edge_index[1], jnp.full((EP - E,), N, jnp.int32)])
  node_ids = jnp.arange(NP, dtype=jnp.int32)
  batch_p = jnp.concatenate([batch, jnp.full((NP - N,), NUM_GRAPHS, jnp.int32)])
  ones_tab = jnp.ones((8, HID), f32)
  zro_n = jnp.zeros((NP, HID), f32)
  zro_g = jnp.zeros((GP, HID), f32)
  zeros_e = jnp.zeros((EP,), jnp.int32)

  sc_edges = _make_sc_scatter(NP, EP, NP, 128)
  sc_deg = _make_sc_scatter(8, EP, NP, 128)
  sc_pool = _make_sc_scatter(NP, NP, GP, 64)
  sc_pcnt = _make_sc_scatter(8, NP, GP, 64)

  h0 = pl.pallas_call(
      _enc_body,
      grid=(NP // TE,),
      in_specs=[pl.BlockSpec((TE, 4, 130), lambda i: (i, 0, 0)),
                _full_spec((7, 16, 128)), _full_spec((16, 128)),
                _full_spec((80, 32, 128)), _full_spec((32, 128)),
                _full_spec((32, HID)), _full_spec((1, HID))],
      out_specs=pl.BlockSpec((TE, HID), lambda i: (i, 0)),
      out_shape=jax.ShapeDtypeStruct((NP, HID), f32),
  )(xq, w1, b1, w2, b2, lin_w, lb)

  dacc = sc_deg(ones_tab, zeros_e, dst_p, zro_n)
  da0, da1 = dacc[:NP], dacc[NP:]

  def gcn_layer(h, W, b2d):
    hws = pl.pallas_call(
        _pre_body,
        grid=(grid,),
        in_specs=[_row_spec(HID), _full_spec((HID, HID)),
                  _row_spec(HID), _row_spec(HID)],
        out_specs=_row_spec(HID),
        out_shape=jax.ShapeDtypeStruct((NP, HID), f32),
    )(h, W, da0, da1)
    sacc = sc_edges(hws, src_p, dst_p, zro_n)
    return pl.pallas_call(
        _post_body,
        grid=(grid,),
        in_specs=[_row_spec(HID), _row_spec(HID), _row_spec(HID),
                  _row_spec(HID), _row_spec(HID), _full_spec((1, HID))],
        out_specs=_row_spec(HID),
        out_shape=jax.ShapeDtypeStruct((NP, HID), f32),
    )(sacc[:NP], sacc[NP:], hws, da0, da1, b2d)

  h1 = gcn_layer(h0, g1_w, g1b)
  h2 = gcn_layer(h1, g2_w, g2b)

  pacc = sc_pool(h2, node_ids, batch_p, zro_g)
  cacc = sc_pcnt(ones_tab, jnp.zeros((NP,), jnp.int32), batch_p, zro_g)
  logits = pl.pallas_call(
      _final_body,
      grid=(1,),
      in_specs=[_full_spec((GP, HID))] * 4 + [_full_spec((HID, HID)),
                                              _full_spec((1, HID))],
      out_specs=_full_spec((GP, HID)),
      out_shape=jax.ShapeDtypeStruct((GP, HID), f32),
  )(pacc[:GP], pacc[GP:], cacc[:GP], cacc[GP:], fw, fb)
  return logits[:NUM_GRAPHS, :fc_w.shape[1]]


# speedup vs baseline: 1.9045x; 1.9045x over previous
import functools

import jax
import jax.numpy as jnp
from jax import lax
from jax.experimental import pallas as pl
from jax.experimental.pallas import tpu as pltpu
from jax.experimental.pallas import tpu_sc as plsc

N = 10000
E = 320000
L = 512
HID = 128
NUM_GRAPHS = 1000

NP = 10240
GP = 1024
TN = 256
NC, NS = 2, 16
NW = NC * NS


def _make_sc_scatter(T, Ep, ACC, CH, D, hoist_gather=False):
  assert Ep % (NW * CH) == 0 and ACC % NS == 0
  nch = Ep // (NW * CH)
  per_w = Ep // NW
  zr = ACC // NS
  mesh = plsc.VectorSubcoreMesh(core_axis_name="c", subcore_axis_name="s")

  @functools.partial(
      pl.kernel, mesh=mesh,
      out_type=jax.ShapeDtypeStruct((2 * ACC, D), jnp.float32),
      scratch_types=[
          pltpu.VMEM((CH,), jnp.int32),
          pltpu.VMEM((CH,), jnp.int32),
          pltpu.VMEM((CH, D), jnp.float32),
          pltpu.VMEM_SHARED((ACC, D), jnp.float32),
          pltpu.SemaphoreType.DMA,
      ],
  )
  def k(table, srci, dsti, zro, out, sidx_v, didx_v, rows_v, acc_sh, sem):
    cid = lax.axis_index("c")
    sid = lax.axis_index("s")
    pltpu.sync_copy(zro.at[pl.ds(sid * zr, zr)], acc_sh.at[pl.ds(sid * zr, zr)])
    plsc.subcore_barrier()
    wid = sid * NC + cid
    base0 = wid * per_w
    if hoist_gather:
      pltpu.sync_copy(table, rows_v)

    def body(i, _):
      base = base0 + i * CH
      if not hoist_gather:
        pltpu.sync_copy(srci.at[pl.ds(base, CH)], sidx_v)
        pltpu.async_copy(table.at[sidx_v], rows_v, sem).wait()
      pltpu.sync_copy(dsti.at[pl.ds(base, CH)], didx_v)
      pltpu.sync_copy(rows_v, acc_sh.at[didx_v], add=True)
      return 0

    lax.fori_loop(0, nch, body, 0)
    plsc.subcore_barrier()
    pltpu.sync_copy(acc_sh.at[pl.ds(sid * zr, zr)],
                    out.at[pl.ds(cid * ACC + sid * zr, zr)])

  return k


TE = 32


def _enc_body(xq_ref, w1_ref, b1_ref, w2_ref, b2_ref, lw_ref, lb_ref, o_ref):
  xq = xq_ref[...]
  w1 = w1_ref[...]
  h1e = jnp.zeros((TE, 16, 128), jnp.float32)
  h1o = jnp.zeros((TE, 16, 128), jnp.float32)
  for kk in range(7):
    wk = w1[kk][None]
    h1e = h1e + xq[:, kk % 4, (kk // 4):(kk // 4) + 128][:, None, :] * wk
    ko = kk + 2
    h1o = h1o + xq[:, ko % 4, (ko // 4):(ko // 4) + 128][:, None, :] * wk
  b1 = b1_ref[...][None]
  h1e = jnp.maximum(h1e + b1, 0.0)
  h1o = jnp.maximum(h1o + b1, 0.0)
  z1 = jnp.zeros((TE, 16, 1), jnp.float32)
  h1pe = jnp.concatenate([z1, h1e, z1], axis=2)
  h1po = jnp.concatenate([z1, h1o, z1], axis=2)
  w2 = w2_ref[...]
  acc = jnp.zeros((TE, 32, 128), jnp.float32)
  for kk in range(5):
    off = kk // 2 if kk % 2 == 0 else (kk - 1) // 2
    hp = h1pe if kk % 2 == 0 else h1po
    for ii in range(16):
      hs = hp[:, ii, off:off + 128]
      acc = acc + hs[:, None, :] * w2[kk * 16 + ii][None]
  acc = jnp.maximum(acc + b2_ref[...][None], 0.0)
  m = jnp.mean(acc, axis=2)
  o_ref[...] = m @ lw_ref[...] + lb_ref[...][0][None, :]


def _pre_body(h_ref, w_ref, a0_ref, a1_ref, o_ref):
  dis = lax.rsqrt(1.0 + a0_ref[...][:, :1] + a1_ref[...][:, :1])
  o_ref[...] = dis * (h_ref[...] @ w_ref[...])


def _post_body(s0_ref, s1_ref, hws_ref, a0_ref, a1_ref, b_ref, o_ref):
  dis = lax.rsqrt(1.0 + a0_ref[...][:, :1] + a1_ref[...][:, :1])
  v = dis * (s0_ref[...] + s1_ref[...] + hws_ref[...]) + b_ref[...][0][None, :]
  o_ref[...] = jnp.maximum(v, 0.0)


def _final_body(p0_ref, p1_ref, c0_ref, c1_ref, fw_ref, fb_ref, o_ref):
  cnt = jnp.maximum(c0_ref[...][:, :1] + c1_ref[...][:, :1], 1.0)
  pooled = (p0_ref[...] + p1_ref[...]) / cnt
  o_ref[...] = pooled @ fw_ref[...] + fb_ref[...][0][None, :]


def _row_spec(d):
  return pl.BlockSpec((TN, d), lambda i: (i, 0))


def _full_spec(shape):
  return pl.BlockSpec(shape, lambda i: tuple(0 for _ in shape))


def kernel(x, edge_index, batch, conv1_w, conv1_b, conv2_w, conv2_b,
           lin_w, lin_b, g1_w, g1_b, g2_w, g2_b, fc_w, fc_b):
  f32 = jnp.float32
  grid = NP // TN

  x_p = jnp.concatenate([x, jnp.zeros((NP - N, L), f32)], axis=0)
  xpad = jnp.pad(x_p, ((0, 0), (3, 5)))
  xq = jnp.transpose(xpad.reshape(NP, 130, 4), (0, 2, 1))
  w1 = jnp.broadcast_to(conv1_w.reshape(16, 7).T[:, :, None], (7, 16, 128))
  w2 = jnp.broadcast_to(
      jnp.transpose(conv2_w, (2, 1, 0)).reshape(80, 32)[:, :, None],
      (80, 32, 128))
  b1 = jnp.broadcast_to(conv1_b[:, None], (16, 128))
  b2 = jnp.broadcast_to(conv2_b[:, None], (32, 128))
  lb = lin_b.reshape(1, HID)
  g1b = g1_b.reshape(1, HID)
  g2b = g2_b.reshape(1, HID)
  fw = jnp.zeros((HID, HID), f32).at[:, :fc_w.shape[1]].set(fc_w)
  fb = jnp.zeros((1, HID), f32).at[0, :fc_b.shape[0]].set(fc_b)

  EP = 323584
  src_p = jnp.concatenate([edge_index[0], jnp.zeros((EP - E,), jnp.int32)])
  dst_p = jnp.concatenate([edge_index[1], jnp.full((EP - E,), N, jnp.int32)])
  node_ids = jnp.arange(NP, dtype=jnp.int32)
  batch_p = jnp.concatenate([batch, jnp.full((NP - N,), NUM_GRAPHS, jnp.int32)])
  ones_128 = jnp.ones((128, HID), f32)
  ones_64 = jnp.ones((64, HID), f32)
  zro_n = jnp.zeros((NP, HID), f32)
  zro_n16 = jnp.zeros((NP, HID), f32)
  zro_g16 = jnp.zeros((GP, HID), f32)
  zro_g = jnp.zeros((GP, HID), f32)
  zeros_e = jnp.zeros((EP,), jnp.int32)

  sc_edges = _make_sc_scatter(NP, EP, NP, 128, HID)
  sc_deg = _make_sc_scatter(128, EP, NP, 128, HID, hoist_gather=True)
  sc_pool = _make_sc_scatter(NP, NP, GP, 64, HID)
  sc_pcnt = _make_sc_scatter(64, NP, GP, 64, HID, hoist_gather=True)

  h0 = pl.pallas_call(
      _enc_body,
      grid=(NP // TE,),
      in_specs=[pl.BlockSpec((TE, 4, 130), lambda i: (i, 0, 0)),
                _full_spec((7, 16, 128)), _full_spec((16, 128)),
                _full_spec((80, 32, 128)), _full_spec((32, 128)),
                _full_spec((32, HID)), _full_spec((1, HID))],
      out_specs=pl.BlockSpec((TE, HID), lambda i: (i, 0)),
      out_shape=jax.ShapeDtypeStruct((NP, HID), f32),
  )(xq, w1, b1, w2, b2, lin_w, lb)

  dacc = sc_deg(ones_128, zeros_e, dst_p, zro_n16)
  da0, da1 = dacc[:NP], dacc[NP:]

  def gcn_layer(h, W, b2d):
    hws = pl.pallas_call(
        _pre_body,
        grid=(grid,),
        in_specs=[_row_spec(HID), _full_spec((HID, HID)),
                  _row_spec(HID), _row_spec(HID)],
        out_specs=_row_spec(HID),
        out_shape=jax.ShapeDtypeStruct((NP, HID), f32),
    )(h, W, da0, da1)
    sacc = sc_edges(hws, src_p, dst_p, zro_n)
    return pl.pallas_call(
        _post_body,
        grid=(grid,),
        in_specs=[_row_spec(HID), _row_spec(HID), _row_spec(HID),
                  _row_spec(HID), _row_spec(HID), _full_spec((1, HID))],
        out_specs=_row_spec(HID),
        out_shape=jax.ShapeDtypeStruct((NP, HID), f32),
    )(sacc[:NP], sacc[NP:], hws, da0, da1, b2d)

  h1 = gcn_layer(h0, g1_w, g1b)
  h2 = gcn_layer(h1, g2_w, g2b)

  pacc = sc_pool(h2, node_ids, batch_p, zro_g)
  cacc = sc_pcnt(ones_64, jnp.zeros((NP,), jnp.int32), batch_p, zro_g16)
  logits = pl.pallas_call(
      _final_body,
      grid=(1,),
      in_specs=[_full_spec((GP, HID))] * 4 +
               [_full_spec((HID, HID)), _full_spec((1, HID))],
      out_specs=_full_spec((GP, HID)),
      out_shape=jax.ShapeDtypeStruct((GP, HID), f32),
  )(pacc[:GP], pacc[GP:], cacc[:GP], cacc[GP:], fw, fb)
  return logits[:NUM_GRAPHS, :fc_w.shape[1]]
